# baseline (speedup 1.0000x reference)
import functools

import jax
import jax.numpy as jnp
from jax import lax
from jax.experimental import pallas as pl
from jax.experimental.pallas import tpu as pltpu
from jax.experimental.pallas import tpu_sc as plsc

Q = 1024
D = 64
N = 100000
KSEL = 10
C = 1024
NBLK = (N + C - 1) // C
NPAD = NBLK * C
G = 128
NG = NPAD // G

NWORK = 32
RPW = Q * KSEL // NWORK
CHROW = 64
NCH = RPW // CHROW

F32_INF = float("inf")
F32_BIG = 3.0e8


def _dist_kernel(rx_ref, tx_ref, d2_ref, ridt_ref, gmt_ref):
    c = pl.program_id(0)
    rxb = rx_ref[...]
    txb = tx_ref[...]
    rx2 = jnp.sum(rxb * rxb, axis=1, keepdims=True)
    tx2 = jnp.sum(txb * txb, axis=1)[None, :]
    prod = lax.dot_general(
        rxb, txb, (((1,), (1,)), ((), ())),
        preferred_element_type=jnp.float32,
    )
    d2 = jnp.maximum(rx2 + tx2 - 2.0 * prod, 0.0)

    @pl.when(c == NBLK - 1)
    def _store_tail():
        gcol = c * C + jax.lax.broadcasted_iota(jnp.int32, (Q, C), 1)
        d2m = jnp.where(gcol < N, d2, F32_INF)
        for g in range(C // G):
            d2_ref[:, g, :] = d2m[:, g * G:(g + 1) * G]

    @pl.when(c < NBLK - 1)
    def _store():
        for g in range(C // G):
            d2_ref[:, g, :] = d2[:, g * G:(g + 1) * G]

    gm8 = jnp.concatenate(
        [jnp.min(d2_ref[:, g, :], axis=1, keepdims=True)
         for g in range(C // G)], axis=1)
    gmt_ref[pl.ds(c * (C // G), C // G), :] = gm8.T

    @pl.when(c == NBLK - 1)
    def _pick():
        tv = gmt_ref[...]
        gio = jax.lax.broadcasted_iota(
            jnp.int32, (NG, Q), 0).astype(jnp.float32)
        qio = jax.lax.broadcasted_iota(jnp.int32, (1, Q), 1)
        for j in range(KSEL):
            mv = jnp.min(tv, axis=0, keepdims=True)
            mg = jnp.min(jnp.where(tv == mv, gio, F32_BIG),
                         axis=0, keepdims=True)
            ridt_ref[pl.ds(j, 1), :] = qio * NG + mg.astype(jnp.int32)
            tv = jnp.where((tv == mv) & (gio == mg), F32_INF, tv)
        ridt_ref[pl.ds(KSEL, 16 - KSEL), :] = jnp.zeros(
            (16 - KSEL, Q), jnp.int32)


def _final_kernel(cand_ref, rid_ref, outd_ref, outi_ref, idx_scr):
    qio = jax.lax.broadcasted_iota(jnp.int32, (Q, 1), 0)
    lio = jax.lax.broadcasted_iota(jnp.int32, (Q, G), 1)
    for j in range(KSEL):
        rid = rid_ref[:, pl.ds(j, 1)]
        gid = rid - qio * NG
        idx_scr[:, pl.ds(j * G, G)] = (gid * G + lio).astype(jnp.float32)
    vals = cand_ref[...]
    gidx = idx_scr[...]
    for j in range(KSEL):
        mv = jnp.min(vals, axis=1, keepdims=True)
        mi = jnp.min(jnp.where(vals == mv, gidx, F32_BIG),
                     axis=1, keepdims=True)
        outd_ref[:, pl.ds(j, 1)] = jnp.sqrt(mv + 1e-12)
        outi_ref[:, pl.ds(j, 1)] = mi.astype(jnp.int32)
        vals = jnp.where((vals == mv) & (gidx == mi), F32_INF, vals)


def _make_gather():
    mesh = plsc.VectorSubcoreMesh(core_axis_name="c", subcore_axis_name="s")

    @functools.partial(
        pl.kernel, mesh=mesh,
        out_type=jax.ShapeDtypeStruct((Q * KSEL, G), jnp.float32),
        scratch_types=[
            pltpu.VMEM((NCH, CHROW), jnp.int32),
            pltpu.VMEM((RPW, G), jnp.float32),
            pltpu.SemaphoreType.DMA,
        ],
    )
    def _gather_kernel(tab_hbm, idx_hbm, out_hbm, idx_v, rows_v, sem):
        wid = lax.axis_index("s") * 2 + lax.axis_index("c")
        pltpu.sync_copy(idx_hbm.at[wid], idx_v)
        copies = [
            pltpu.async_copy(
                tab_hbm.at[idx_v.at[i]],
                rows_v.at[pl.ds(i * CHROW, CHROW)], sem)
            for i in range(NCH)
        ]
        for cp in copies:
            cp.wait()
        pltpu.sync_copy(rows_v, out_hbm.at[pl.ds(wid * RPW, RPW)])

    return _gather_kernel


@jax.jit
def _knn(rx, tx):
    d2, ridt = pl.pallas_call(
        _dist_kernel,
        grid=(NBLK,),
        in_specs=[
            pl.BlockSpec((Q, D), lambda c: (0, 0)),
            pl.BlockSpec((C, D), lambda c: (c, 0)),
        ],
        out_specs=[
            pl.BlockSpec((Q, C // G, G), lambda c: (0, c, 0)),
            pl.BlockSpec((16, Q), lambda c: (0, 0)),
        ],
        out_shape=[
            jax.ShapeDtypeStruct((Q, NG, G), jnp.float32),
            jax.ShapeDtypeStruct((16, Q), jnp.int32),
        ],
        scratch_shapes=[pltpu.VMEM((NG, Q), jnp.float32)],
    )(rx, tx)

    rid_qmaj = ridt[:KSEL, :].T
    idx3 = rid_qmaj.reshape(NWORK, NCH, CHROW)
    tab = d2.reshape(Q * NG, G)
    cand = _make_gather()(tab, idx3).reshape(Q, KSEL * G)

    rid2 = jnp.pad(rid_qmaj, ((0, 0), (0, 16 - KSEL)))
    outd, outi = pl.pallas_call(
        _final_kernel,
        out_shape=[
            jax.ShapeDtypeStruct((Q, 128), jnp.float32),
            jax.ShapeDtypeStruct((Q, 128), jnp.int32),
        ],
        scratch_shapes=[pltpu.VMEM((Q, KSEL * G), jnp.float32)],
    )(cand, rid2)
    return outi[:, :KSEL].reshape(-1), outd[:, :KSEL].reshape(-1)


def kernel(rx, tx, k):
    del k
    return _knn(rx, tx)


# speedup vs baseline: 1.2238x; 1.2238x over previous
import functools

import jax
import jax.numpy as jnp
from jax import lax
from jax.experimental import pallas as pl
from jax.experimental.pallas import tpu as pltpu
from jax.experimental.pallas import tpu_sc as plsc

Q = 1024
D = 64
N = 100000
KSEL = 10
C = 1024
NBLK = (N + C - 1) // C
NPAD = NBLK * C
G = 128
NG = NPAD // G

NWORK = 32
RPW = Q * KSEL // NWORK
CHROW = 64
NCH = RPW // CHROW

F32_INF = float("inf")
F32_BIG = 3.0e8


def _dist_kernel(rx_ref, tx_ref, d2_ref, ridt_ref, gmt_ref):
    c = pl.program_id(0)
    rxb = rx_ref[...]
    txb = tx_ref[...]
    rx2 = jnp.sum(rxb * rxb, axis=1, keepdims=True)
    tx2 = jnp.sum(txb * txb, axis=1)[None, :]
    prod = lax.dot_general(
        rxb, txb, (((1,), (1,)), ((), ())),
        preferred_element_type=jnp.float32,
    )
    d2 = jnp.maximum(rx2 + tx2 - 2.0 * prod, 0.0)

    def _store_fold(dvals):
        for g in range(C // G):
            d2_ref[:, g, :] = dvals[:, g * G:(g + 1) * G]
        gm8 = jnp.concatenate(
            [jnp.min(dvals[:, g * G:(g + 1) * G], axis=1, keepdims=True)
             for g in range(C // G)], axis=1)
        gmt_ref[pl.ds(c * (C // G), C // G), :] = gm8.T

    @pl.when(c < NBLK - 1)
    def _main():
        _store_fold(d2)

    @pl.when(c == NBLK - 1)
    def _tail():
        gcol = c * C + jax.lax.broadcasted_iota(jnp.int32, (Q, C), 1)
        _store_fold(jnp.where(gcol < N, d2, F32_INF))

    @pl.when(c == NBLK - 1)
    def _pick():
        tv = gmt_ref[...]
        gio = jax.lax.broadcasted_iota(
            jnp.int32, (NG, Q), 0).astype(jnp.float32)
        qio = jax.lax.broadcasted_iota(jnp.int32, (1, Q), 1)
        for j in range(KSEL):
            mv = jnp.min(tv, axis=0, keepdims=True)
            mg = jnp.min(jnp.where(tv == mv, gio, F32_BIG),
                         axis=0, keepdims=True)
            ridt_ref[pl.ds(j, 1), :] = qio * NG + mg.astype(jnp.int32)
            tv = jnp.where((tv == mv) & (gio == mg), F32_INF, tv)
        ridt_ref[pl.ds(KSEL, 16 - KSEL), :] = jnp.zeros(
            (16 - KSEL, Q), jnp.int32)


def _final_kernel(cand_ref, rid_ref, outd_ref, outi_ref, idx_scr):
    qio = jax.lax.broadcasted_iota(jnp.int32, (Q, 1), 0)
    lio = jax.lax.broadcasted_iota(jnp.int32, (Q, G), 1)
    for j in range(KSEL):
        rid = rid_ref[:, pl.ds(j, 1)]
        gid = rid - qio * NG
        idx_scr[:, pl.ds(j * G, G)] = (gid * G + lio).astype(jnp.float32)
    vals = cand_ref[...]
    gidx = idx_scr[...]
    for j in range(KSEL):
        mv = jnp.min(vals, axis=1, keepdims=True)
        mi = jnp.min(jnp.where(vals == mv, gidx, F32_BIG),
                     axis=1, keepdims=True)
        outd_ref[:, pl.ds(j, 1)] = jnp.sqrt(mv + 1e-12)
        outi_ref[:, pl.ds(j, 1)] = mi.astype(jnp.int32)
        vals = jnp.where((vals == mv) & (gidx == mi), F32_INF, vals)


def _make_gather():
    mesh = plsc.VectorSubcoreMesh(core_axis_name="c", subcore_axis_name="s")

    @functools.partial(
        pl.kernel, mesh=mesh,
        out_type=jax.ShapeDtypeStruct((Q * KSEL, G), jnp.float32),
        scratch_types=[
            pltpu.VMEM((NCH, CHROW), jnp.int32),
            pltpu.VMEM((RPW, G), jnp.float32),
            pltpu.SemaphoreType.DMA,
        ],
    )
    def _gather_kernel(tab_hbm, idx_hbm, out_hbm, idx_v, rows_v, sem):
        wid = lax.axis_index("s") * 2 + lax.axis_index("c")
        pltpu.sync_copy(idx_hbm.at[wid], idx_v)
        copies = [
            pltpu.async_copy(
                tab_hbm.at[idx_v.at[i]],
                rows_v.at[pl.ds(i * CHROW, CHROW)], sem)
            for i in range(NCH)
        ]
        for cp in copies:
            cp.wait()
        pltpu.sync_copy(rows_v, out_hbm.at[pl.ds(wid * RPW, RPW)])

    return _gather_kernel


@jax.jit
def _knn(rx, tx):
    d2, ridt = pl.pallas_call(
        _dist_kernel,
        grid=(NBLK,),
        in_specs=[
            pl.BlockSpec((Q, D), lambda c: (0, 0)),
            pl.BlockSpec((C, D), lambda c: (c, 0)),
        ],
        out_specs=[
            pl.BlockSpec((Q, C // G, G), lambda c: (0, c, 0)),
            pl.BlockSpec((16, Q), lambda c: (0, 0)),
        ],
        out_shape=[
            jax.ShapeDtypeStruct((Q, NG, G), jnp.float32),
            jax.ShapeDtypeStruct((16, Q), jnp.int32),
        ],
        scratch_shapes=[pltpu.VMEM((NG, Q), jnp.float32)],
    )(rx, tx)

    rid_qmaj = ridt[:KSEL, :].T
    idx3 = rid_qmaj.reshape(NWORK, NCH, CHROW)
    tab = d2.reshape(Q * NG, G)
    cand = _make_gather()(tab, idx3).reshape(Q, KSEL * G)

    rid2 = jnp.pad(rid_qmaj, ((0, 0), (0, 16 - KSEL)))
    outd, outi = pl.pallas_call(
        _final_kernel,
        out_shape=[
            jax.ShapeDtypeStruct((Q, 128), jnp.float32),
            jax.ShapeDtypeStruct((Q, 128), jnp.int32),
        ],
        scratch_shapes=[pltpu.VMEM((Q, KSEL * G), jnp.float32)],
    )(cand, rid2)
    return outi[:, :KSEL].reshape(-1), outd[:, :KSEL].reshape(-1)


def kernel(rx, tx, k):
    del k
    return _knn(rx, tx)


# speedup vs baseline: 1.8801x; 1.5364x over previous
import functools

import jax
import jax.numpy as jnp
from jax import lax
from jax.experimental import pallas as pl
from jax.experimental.pallas import tpu as pltpu
from jax.experimental.pallas import tpu_sc as plsc

Q = 1024
D = 64
N = 100000
KSEL = 10
C = 1024
NBLK = (N + C - 1) // C
NPAD = NBLK * C
G = 128
NG = NPAD // G

NWORK = 32
RPW = Q * KSEL // NWORK
CHROW = 64
NCH = RPW // CHROW

F32_INF = float("inf")
F32_BIG = 3.0e8


def _dist_kernel(rx_ref, tx_ref, d2_ref, ridt_ref, gmt_ref):
    c = pl.program_id(0)
    rxb = rx_ref[...]
    txb = tx_ref[...]
    rx2 = jnp.sum(rxb * rxb, axis=1, keepdims=True)
    tx2 = jnp.sum(txb * txb, axis=1)[None, :]
    prod = lax.dot_general(
        rxb, txb, (((1,), (1,)), ((), ())),
        preferred_element_type=jnp.float32,
    )
    d2 = jnp.maximum(rx2 + tx2 - 2.0 * prod, 0.0)

    def _store_fold(dvals):
        for g in range(C // G):
            d2_ref[g, :, :] = dvals[:, g * G:(g + 1) * G]
        gm8 = jnp.concatenate(
            [jnp.min(dvals[:, g * G:(g + 1) * G], axis=1, keepdims=True)
             for g in range(C // G)], axis=1)
        gmt_ref[pl.ds(c * (C // G), C // G), :] = gm8.T

    @pl.when(c < NBLK - 1)
    def _main():
        _store_fold(d2)

    @pl.when(c == NBLK - 1)
    def _tail():
        gcol = c * C + jax.lax.broadcasted_iota(jnp.int32, (Q, C), 1)
        _store_fold(jnp.where(gcol < N, d2, F32_INF))

    @pl.when(c == NBLK - 1)
    def _pick():
        tv = gmt_ref[...]
        gio = jax.lax.broadcasted_iota(
            jnp.int32, (NG, Q), 0).astype(jnp.float32)
        qio = jax.lax.broadcasted_iota(jnp.int32, (1, Q), 1)
        for j in range(KSEL):
            mv = jnp.min(tv, axis=0, keepdims=True)
            mg = jnp.min(jnp.where(tv == mv, gio, F32_BIG),
                         axis=0, keepdims=True)
            ridt_ref[pl.ds(j, 1), :] = mg.astype(jnp.int32) * Q + qio
            tv = jnp.where((tv == mv) & (gio == mg), F32_INF, tv)
        ridt_ref[pl.ds(KSEL, 16 - KSEL), :] = jnp.zeros(
            (16 - KSEL, Q), jnp.int32)


def _final_kernel(cand_ref, rid_ref, outd_ref, outi_ref, idx_scr):
    qio = jax.lax.broadcasted_iota(jnp.int32, (Q, 1), 0)
    lio = jax.lax.broadcasted_iota(jnp.int32, (Q, G), 1)
    for j in range(KSEL):
        rid = rid_ref[:, pl.ds(j, 1)]
        gid = lax.shift_right_logical(rid - qio, 10)
        idx_scr[:, pl.ds(j * G, G)] = (gid * G + lio).astype(jnp.float32)
    vals = cand_ref[...]
    gidx = idx_scr[...]
    for j in range(KSEL):
        mv = jnp.min(vals, axis=1, keepdims=True)
        mi = jnp.min(jnp.where(vals == mv, gidx, F32_BIG),
                     axis=1, keepdims=True)
        outd_ref[:, pl.ds(j, 1)] = jnp.sqrt(mv + 1e-12)
        outi_ref[:, pl.ds(j, 1)] = mi.astype(jnp.int32)
        vals = jnp.where((vals == mv) & (gidx == mi), F32_INF, vals)


def _make_gather():
    mesh = plsc.VectorSubcoreMesh(core_axis_name="c", subcore_axis_name="s")

    @functools.partial(
        pl.kernel, mesh=mesh,
        out_type=jax.ShapeDtypeStruct((Q * KSEL, G), jnp.float32),
        scratch_types=[
            pltpu.VMEM((NCH, CHROW), jnp.int32),
            pltpu.VMEM((RPW, G), jnp.float32),
            pltpu.SemaphoreType.DMA,
        ],
    )
    def _gather_kernel(tab_hbm, idx_hbm, out_hbm, idx_v, rows_v, sem):
        wid = lax.axis_index("s") * 2 + lax.axis_index("c")
        pltpu.sync_copy(idx_hbm.at[wid], idx_v)
        copies = [
            pltpu.async_copy(
                tab_hbm.at[idx_v.at[i]],
                rows_v.at[pl.ds(i * CHROW, CHROW)], sem)
            for i in range(NCH)
        ]
        for cp in copies:
            cp.wait()
        pltpu.sync_copy(rows_v, out_hbm.at[pl.ds(wid * RPW, RPW)])

    return _gather_kernel


@jax.jit
def _knn(rx, tx):
    d2, ridt = pl.pallas_call(
        _dist_kernel,
        grid=(NBLK,),
        in_specs=[
            pl.BlockSpec((Q, D), lambda c: (0, 0)),
            pl.BlockSpec((C, D), lambda c: (c, 0)),
        ],
        out_specs=[
            pl.BlockSpec((C // G, Q, G), lambda c: (c, 0, 0)),
            pl.BlockSpec((16, Q), lambda c: (0, 0)),
        ],
        out_shape=[
            jax.ShapeDtypeStruct((NG, Q, G), jnp.float32),
            jax.ShapeDtypeStruct((16, Q), jnp.int32),
        ],
        scratch_shapes=[pltpu.VMEM((NG, Q), jnp.float32)],
    )(rx, tx)

    rid_qmaj = ridt[:KSEL, :].T
    idx3 = rid_qmaj.reshape(NWORK, NCH, CHROW)
    tab = d2.reshape(NG * Q, G)
    cand = _make_gather()(tab, idx3).reshape(Q, KSEL * G)

    rid2 = jnp.pad(rid_qmaj, ((0, 0), (0, 16 - KSEL)))
    outd, outi = pl.pallas_call(
        _final_kernel,
        out_shape=[
            jax.ShapeDtypeStruct((Q, 128), jnp.float32),
            jax.ShapeDtypeStruct((Q, 128), jnp.int32),
        ],
        scratch_shapes=[pltpu.VMEM((Q, KSEL * G), jnp.float32)],
    )(cand, rid2)
    return outi[:, :KSEL].reshape(-1), outd[:, :KSEL].reshape(-1)


def kernel(rx, tx, k):
    del k
    return _knn(rx, tx)


# speedup vs baseline: 1.9959x; 1.0616x over previous
import functools

import jax
import jax.numpy as jnp
from jax import lax
from jax.experimental import pallas as pl
from jax.experimental.pallas import tpu as pltpu
from jax.experimental.pallas import tpu_sc as plsc

Q = 1024
D = 64
N = 100000
KSEL = 10
C = 2048
NBLK = (N + C - 1) // C
NPAD = NBLK * C
G = 128
NG = NPAD // G

NWORK = 32
RPW = Q * KSEL // NWORK
CHROW = 64
NCH = RPW // CHROW

F32_INF = float("inf")
F32_BIG = 3.0e8


def _dist_kernel(rx_ref, tx_ref, d2_ref, ridt_ref, gmt_ref):
    c = pl.program_id(0)
    rxb = rx_ref[...]
    txb = tx_ref[...]
    rx2 = jnp.sum(rxb * rxb, axis=1, keepdims=True)
    tx2 = jnp.sum(txb * txb, axis=1)[None, :]
    prod = lax.dot_general(
        rxb, txb, (((1,), (1,)), ((), ())),
        preferred_element_type=jnp.float32,
    )
    d2 = jnp.maximum(rx2 + tx2 - 2.0 * prod, 0.0)

    def _store_fold(dvals):
        for g in range(C // G):
            d2_ref[g, :, :] = dvals[:, g * G:(g + 1) * G]
        gm8 = jnp.concatenate(
            [jnp.min(dvals[:, g * G:(g + 1) * G], axis=1, keepdims=True)
             for g in range(C // G)], axis=1)
        gmt_ref[pl.ds(c * (C // G), C // G), :] = gm8.T

    @pl.when(c < NBLK - 1)
    def _main():
        _store_fold(d2)

    @pl.when(c == NBLK - 1)
    def _tail():
        gcol = c * C + jax.lax.broadcasted_iota(jnp.int32, (Q, C), 1)
        _store_fold(jnp.where(gcol < N, d2, F32_INF))

    @pl.when(c == NBLK - 1)
    def _pick():
        tv = gmt_ref[...]
        gio = jax.lax.broadcasted_iota(
            jnp.int32, (NG, Q), 0).astype(jnp.float32)
        qio = jax.lax.broadcasted_iota(jnp.int32, (1, Q), 1)
        for j in range(KSEL):
            mv = jnp.min(tv, axis=0, keepdims=True)
            mg = jnp.min(jnp.where(tv == mv, gio, F32_BIG),
                         axis=0, keepdims=True)
            ridt_ref[pl.ds(j, 1), :] = mg.astype(jnp.int32) * Q + qio
            tv = jnp.where((tv == mv) & (gio == mg), F32_INF, tv)
        ridt_ref[pl.ds(KSEL, 16 - KSEL), :] = jnp.zeros(
            (16 - KSEL, Q), jnp.int32)


def _final_kernel(cand_ref, rid_ref, outd_ref, outi_ref, idx_scr):
    qio = jax.lax.broadcasted_iota(jnp.int32, (Q, 1), 0)
    lio = jax.lax.broadcasted_iota(jnp.int32, (Q, G), 1)
    for j in range(KSEL):
        rid = rid_ref[:, pl.ds(j, 1)]
        gid = lax.shift_right_logical(rid - qio, 10)
        idx_scr[:, pl.ds(j * G, G)] = (gid * G + lio).astype(jnp.float32)
    vals = cand_ref[...]
    gidx = idx_scr[...]
    for j in range(KSEL):
        mv = jnp.min(vals, axis=1, keepdims=True)
        mi = jnp.min(jnp.where(vals == mv, gidx, F32_BIG),
                     axis=1, keepdims=True)
        outd_ref[:, pl.ds(j, 1)] = jnp.sqrt(mv + 1e-12)
        outi_ref[:, pl.ds(j, 1)] = mi.astype(jnp.int32)
        vals = jnp.where((vals == mv) & (gidx == mi), F32_INF, vals)


def _make_gather():
    mesh = plsc.VectorSubcoreMesh(core_axis_name="c", subcore_axis_name="s")

    @functools.partial(
        pl.kernel, mesh=mesh,
        out_type=jax.ShapeDtypeStruct((Q * KSEL, G), jnp.float32),
        scratch_types=[
            pltpu.VMEM((NCH, CHROW), jnp.int32),
            pltpu.VMEM((RPW, G), jnp.float32),
            pltpu.SemaphoreType.DMA,
        ],
    )
    def _gather_kernel(tab_hbm, idx_hbm, out_hbm, idx_v, rows_v, sem):
        wid = lax.axis_index("s") * 2 + lax.axis_index("c")
        pltpu.sync_copy(idx_hbm.at[wid], idx_v)
        copies = [
            pltpu.async_copy(
                tab_hbm.at[idx_v.at[i]],
                rows_v.at[pl.ds(i * CHROW, CHROW)], sem)
            for i in range(NCH)
        ]
        for cp in copies:
            cp.wait()
        pltpu.sync_copy(rows_v, out_hbm.at[pl.ds(wid * RPW, RPW)])

    return _gather_kernel


@jax.jit
def _knn(rx, tx):
    d2, ridt = pl.pallas_call(
        _dist_kernel,
        grid=(NBLK,),
        in_specs=[
            pl.BlockSpec((Q, D), lambda c: (0, 0)),
            pl.BlockSpec((C, D), lambda c: (c, 0)),
        ],
        out_specs=[
            pl.BlockSpec((C // G, Q, G), lambda c: (c, 0, 0)),
            pl.BlockSpec((16, Q), lambda c: (0, 0)),
        ],
        out_shape=[
            jax.ShapeDtypeStruct((NG, Q, G), jnp.float32),
            jax.ShapeDtypeStruct((16, Q), jnp.int32),
        ],
        scratch_shapes=[pltpu.VMEM((NG, Q), jnp.float32)],
    )(rx, tx)

    rid_qmaj = ridt[:KSEL, :].T
    idx3 = rid_qmaj.reshape(NWORK, NCH, CHROW)
    tab = d2.reshape(NG * Q, G)
    cand = _make_gather()(tab, idx3).reshape(Q, KSEL * G)

    rid2 = jnp.pad(rid_qmaj, ((0, 0), (0, 16 - KSEL)))
    outd, outi = pl.pallas_call(
        _final_kernel,
        out_shape=[
            jax.ShapeDtypeStruct((Q, 128), jnp.float32),
            jax.ShapeDtypeStruct((Q, 128), jnp.int32),
        ],
        scratch_shapes=[pltpu.VMEM((Q, KSEL * G), jnp.float32)],
    )(cand, rid2)
    return outi[:, :KSEL].reshape(-1), outd[:, :KSEL].reshape(-1)


def kernel(rx, tx, k):
    del k
    return _knn(rx, tx)


# speedup vs baseline: 1.9966x; 1.0003x over previous
import functools

import jax
import jax.numpy as jnp
from jax import lax
from jax.experimental import pallas as pl
from jax.experimental.pallas import tpu as pltpu
from jax.experimental.pallas import tpu_sc as plsc

Q = 1024
D = 64
N = 100000
KSEL = 10
C = 2048
NBLK = (N + C - 1) // C
NPAD = NBLK * C
G = 128
NG = NPAD // G

NWORK = 32
RPW = Q * KSEL // NWORK
CHROW = 64
NCH = RPW // CHROW

F32_INF = float("inf")
F32_BIG = 3.0e8


def _dist_kernel(rx_ref, tx_ref, d2_ref, ridt_ref, gmt_ref):
    c = pl.program_id(0)
    rxb = rx_ref[...]
    txb = tx_ref[...]
    rx2 = jnp.sum(rxb * rxb, axis=1, keepdims=True)
    tx2 = jnp.sum(txb * txb, axis=1)[None, :]
    prod = lax.dot_general(
        rxb, txb, (((1,), (1,)), ((), ())),
        preferred_element_type=jnp.float32,
    )
    d2 = jnp.maximum(rx2 + tx2 - 2.0 * prod, 0.0)

    def _store_fold(dvals):
        for g in range(C // G):
            d2_ref[g, :, :] = dvals[:, g * G:(g + 1) * G]
        gm = jnp.concatenate(
            [jnp.min(dvals[:, g * G:(g + 1) * G], axis=1, keepdims=True)
             for g in range(C // G)], axis=1)
        gmt_ref[pl.ds(c * (C // G), C // G), :] = gm.T

    @pl.when(c < NBLK - 1)
    def _main():
        _store_fold(d2)

    @pl.when(c == NBLK - 1)
    def _tail():
        gcol = c * C + jax.lax.broadcasted_iota(jnp.int32, (Q, C), 1)
        _store_fold(jnp.where(gcol < N, d2, F32_INF))

    @pl.when(c == NBLK - 1)
    def _pick():
        tv = gmt_ref[...]
        gio = jax.lax.broadcasted_iota(
            jnp.int32, (NG, Q), 0).astype(jnp.float32)
        qio = jax.lax.broadcasted_iota(jnp.int32, (1, Q), 1)
        for j in range(KSEL):
            mv = jnp.min(tv, axis=0, keepdims=True)
            mg = jnp.min(jnp.where(tv == mv, gio, F32_BIG),
                         axis=0, keepdims=True)
            ridt_ref[pl.ds(j, 1), :] = mg.astype(jnp.int32) * Q + qio
            tv = jnp.where((tv == mv) & (gio == mg), F32_INF, tv)
        ridt_ref[pl.ds(KSEL, 16 - KSEL), :] = jnp.zeros(
            (16 - KSEL, Q), jnp.int32)


def _final_kernel(cand_ref, rid_ref, outd_ref, outi_ref, idx_scr):
    qio = jax.lax.broadcasted_iota(jnp.int32, (Q, 1), 0)
    lio = jax.lax.broadcasted_iota(jnp.int32, (Q, G), 1)
    for j in range(KSEL):
        rid = rid_ref[:, pl.ds(j, 1)]
        gid = lax.shift_right_logical(rid - qio, 10)
        idx_scr[:, pl.ds(j * G, G)] = (gid * G + lio).astype(jnp.float32)
    vals = cand_ref[...]
    gidx = idx_scr[...]
    for j in range(KSEL):
        mv = jnp.min(vals, axis=1, keepdims=True)
        mi = jnp.min(jnp.where(vals == mv, gidx, F32_BIG),
                     axis=1, keepdims=True)
        outd_ref[:, pl.ds(j, 1)] = jnp.sqrt(mv + 1e-12)
        outi_ref[:, pl.ds(j, 1)] = mi.astype(jnp.int32)
        vals = jnp.where((vals == mv) & (gidx == mi), F32_INF, vals)


def _make_gather():
    mesh = plsc.VectorSubcoreMesh(core_axis_name="c", subcore_axis_name="s")

    @functools.partial(
        pl.kernel, mesh=mesh,
        out_type=jax.ShapeDtypeStruct((Q * KSEL, G), jnp.float32),
        scratch_types=[
            pltpu.VMEM((NCH, CHROW), jnp.int32),
            pltpu.VMEM((RPW, G), jnp.float32),
            pltpu.SemaphoreType.DMA,
        ],
    )
    def _gather_kernel(tab_hbm, idx_hbm, out_hbm, idx_v, rows_v, sem):
        wid = lax.axis_index("s") * 2 + lax.axis_index("c")
        pltpu.sync_copy(idx_hbm.at[wid], idx_v)
        copies = [
            pltpu.async_copy(
                tab_hbm.at[idx_v.at[i]],
                rows_v.at[pl.ds(i * CHROW, CHROW)], sem)
            for i in range(NCH)
        ]
        for cp in copies:
            cp.wait()
        pltpu.sync_copy(rows_v, out_hbm.at[pl.ds(wid * RPW, RPW)])

    return _gather_kernel


@jax.jit
def _knn(rx, tx):
    d2, ridt = pl.pallas_call(
        _dist_kernel,
        grid=(NBLK,),
        in_specs=[
            pl.BlockSpec((Q, D), lambda c: (0, 0)),
            pl.BlockSpec((C, D), lambda c: (c, 0)),
        ],
        out_specs=[
            pl.BlockSpec((C // G, Q, G), lambda c: (c, 0, 0)),
            pl.BlockSpec((16, Q), lambda c: (0, 0)),
        ],
        out_shape=[
            jax.ShapeDtypeStruct((NG, Q, G), jnp.float32),
            jax.ShapeDtypeStruct((16, Q), jnp.int32),
        ],
        scratch_shapes=[pltpu.VMEM((NG, Q), jnp.float32)],
    )(rx, tx)

    rid_qmaj = ridt[:KSEL, :].T
    idx3 = rid_qmaj.reshape(NWORK, NCH, CHROW)
    tab = d2.reshape(NG * Q, G)
    cand = _make_gather()(tab, idx3).reshape(Q, KSEL * G)

    rid2 = jnp.pad(rid_qmaj, ((0, 0), (0, 16 - KSEL)))
    outd, outi = pl.pallas_call(
        _final_kernel,
        out_shape=[
            jax.ShapeDtypeStruct((Q, 128), jnp.float32),
            jax.ShapeDtypeStruct((Q, 128), jnp.int32),
        ],
        scratch_shapes=[pltpu.VMEM((Q, KSEL * G), jnp.float32)],
    )(cand, rid2)
    return outi[:, :KSEL].reshape(-1), outd[:, :KSEL].reshape(-1)


def kernel(rx, tx, k):
    del k
    return _knn(rx, tx)
